# baseline (speedup 1.0000x reference)
import functools

import jax
import jax.numpy as jnp
from jax import lax
from jax.experimental import pallas as pl
from jax.experimental.pallas import tpu as pltpu
from jax.experimental.pallas import tpu_sc as plsc

_NC = 2
_NS = 16
_CH = 128


def _build_table_body(cm_ref, w_ref, t_ref):
    Wm = w_ref[:]
    V, D = Wm.shape
    cm = cm_ref[:]
    vv = lax.broadcasted_iota(jnp.int32, (V, V), 1)
    onehot = (cm == vv).astype(jnp.float32)
    wrc = jnp.dot(onehot, Wm, preferred_element_type=jnp.float32,
                  precision=lax.Precision.HIGHEST)
    ii = lax.broadcasted_iota(jnp.int32, (D, D), 0)
    jj = lax.broadcasted_iota(jnp.int32, (D, D), 1)
    rev = (ii + jj == D - 1).astype(jnp.float32)
    t_ref[:, 0:D] = Wm
    t_ref[:, D:2 * D] = jnp.dot(wrc, rev, preferred_element_type=jnp.float32,
                                precision=lax.Precision.HIGHEST)


def kernel(input_ids, W, complement_map):
    Bb, S = input_ids.shape
    V, D = W.shape
    NT = Bb * S
    NW = _NC * _NS
    TPW = NT // NW
    NCH = TPW // _CH

    table = pl.pallas_call(
        _build_table_body,
        out_shape=jax.ShapeDtypeStruct((V, 2 * D), jnp.float32),
    )(complement_map.reshape(V, 1), W)

    ids_flat = input_ids.reshape(NT)
    table_flat = table.reshape(V * 2 * D)

    mesh = plsc.VectorSubcoreMesh(
        core_axis_name="c", subcore_axis_name="s",
        num_cores=_NC, num_subcores=_NS)

    NG = _CH // 16
    ROW = 2 * D

    @functools.partial(
        pl.kernel,
        out_type=jax.ShapeDtypeStruct((NT * ROW,), jnp.float32),
        mesh=mesh,
        scratch_types=[
            pltpu.VMEM((V * ROW,), jnp.float32),
            pltpu.VMEM((TPW,), jnp.int32),
            pltpu.VMEM((_CH * ROW,), jnp.float32),
            pltpu.VMEM((_CH * ROW,), jnp.float32),
            pltpu.SemaphoreType.DMA,
            pltpu.SemaphoreType.DMA,
        ],
        compiler_params=pltpu.CompilerParams(needs_layout_passes=False),
    )
    def sc_embed(t_hbm, ids_hbm, out_hbm, t_v, ids_v, buf0, buf1, s0, s1):
        c = lax.axis_index("c")
        sb = lax.axis_index("s")
        wid = sb * _NC + c
        base = wid * TPW
        pltpu.sync_copy(t_hbm, t_v)
        pltpu.sync_copy(ids_hbm.at[pl.ds(base, TPW)], ids_v)

        bufs = (buf0, buf1)
        ssem = (s0, s1)
        lane = lax.iota(jnp.int32, 16)

        def chunk_compute(g, buf):
            for i in range(NG):
                tok = ids_v[pl.ds(g * _CH + i * 16, 16)]
                rowb = tok * ROW
                sbase = lane * ROW + i * 16 * ROW

                def col_body(col, carry):
                    vec = plsc.load_gather(t_v, [rowb + col])
                    plsc.store_scatter(buf, [sbase + col], vec)
                    return carry

                lax.fori_loop(0, ROW, col_body, 0, unroll=8)

        def start_scatter(g):
            return pltpu.async_copy(
                bufs[g % 2],
                out_hbm.at[pl.ds((base + g * _CH) * ROW, _CH * ROW)],
                ssem[g % 2])

        scatters = [None] * NCH
        for g in range(NCH):
            if g >= 2:
                scatters[g - 2].wait()
            chunk_compute(g, bufs[g % 2])
            scatters[g] = start_scatter(g)
        for g in range(max(0, NCH - 2), NCH):
            scatters[g].wait()

    out = sc_embed(table_flat, ids_flat)
    return out.reshape(Bb, S, 2 * D)


# speedup vs baseline: 6.2051x; 6.2051x over previous
import functools

import jax
import jax.numpy as jnp
from jax import lax
from jax.experimental import pallas as pl
from jax.experimental.pallas import tpu as pltpu
from jax.experimental.pallas import tpu_sc as plsc

_NC = 2
_NS = 16
_CH = 128
_NB = 3


def _build_table_body(cm_ref, w_ref, t_ref):
    Wm = w_ref[:]
    V, D = Wm.shape
    cm = cm_ref[:]
    vv = lax.broadcasted_iota(jnp.int32, (V, V), 1)
    onehot = (cm == vv).astype(jnp.float32)
    wrc = jnp.dot(onehot, Wm, preferred_element_type=jnp.float32,
                  precision=lax.Precision.HIGHEST)
    ii = lax.broadcasted_iota(jnp.int32, (D, D), 0)
    jj = lax.broadcasted_iota(jnp.int32, (D, D), 1)
    rev = (ii + jj == D - 1).astype(jnp.float32)
    rcrev = jnp.dot(wrc, rev, preferred_element_type=jnp.float32,
                    precision=lax.Precision.HIGHEST)
    NW = t_ref.shape[0]
    t_ref[:, :, 0:D] = jnp.broadcast_to(Wm, (NW, V, D))
    t_ref[:, :, D:2 * D] = jnp.broadcast_to(rcrev, (NW, V, D))


def kernel(input_ids, W, complement_map):
    Bb, S = input_ids.shape
    V, D = W.shape
    NT = Bb * S
    NW = _NC * _NS
    TPW = NT // NW
    NCH = TPW // _CH
    ROW = 2 * D

    table_rep = pl.pallas_call(
        _build_table_body,
        out_shape=jax.ShapeDtypeStruct((NW, V, ROW), jnp.float32),
    )(complement_map.reshape(V, 1), W)

    ids_flat = input_ids.reshape(NT)
    table_flat = table_rep.reshape(NW * V, ROW)

    mesh = plsc.VectorSubcoreMesh(
        core_axis_name="c", subcore_axis_name="s",
        num_cores=_NC, num_subcores=_NS)

    @functools.partial(
        pl.kernel,
        out_type=jax.ShapeDtypeStruct((NT, ROW), jnp.float32),
        mesh=mesh,
        scratch_types=[
            pltpu.VMEM((TPW,), jnp.int32),
            [pltpu.VMEM((_CH, ROW), jnp.float32)] * _NB,
            [pltpu.SemaphoreType.DMA] * _NB,
            [pltpu.SemaphoreType.DMA] * _NB,
        ],
    )
    def sc_embed(t_hbm, ids_hbm, out_hbm, idx_v, bufs, gsem, ssem):
        c = lax.axis_index("c")
        sb = lax.axis_index("s")
        wid = sb * _NC + c
        base = wid * TPW
        pltpu.sync_copy(ids_hbm.at[pl.ds(base, TPW)], idx_v)
        off = wid * V
        for i in range(TPW // 16):
            sl = pl.ds(i * 16, 16)
            idx_v[sl] = idx_v[sl] + off

        def start_gather(g):
            return pltpu.async_copy(
                t_hbm.at[idx_v.at[pl.ds(g * _CH, _CH)]],
                bufs[g % _NB], gsem[g % _NB])

        def start_scatter(g):
            return pltpu.async_copy(
                bufs[g % _NB], out_hbm.at[pl.ds(base + g * _CH, _CH)],
                ssem[g % _NB])

        gathers = [None] * NCH
        scatters = [None] * NCH
        for g in range(min(_NB - 1, NCH)):
            gathers[g] = start_gather(g)
        for g in range(NCH):
            gathers[g].wait()
            scatters[g] = start_scatter(g)
            n = g + _NB - 1
            if n < NCH:
                if g >= 1:
                    scatters[g - 1].wait()
                gathers[n] = start_gather(n)
        for g in range(max(0, NCH - _NB), NCH):
            scatters[g].wait()

    out = sc_embed(table_flat, ids_flat)
    return out.reshape(Bb, S, ROW)
